# baseline (speedup 1.0000x reference)
import functools

import jax
import jax.numpy as jnp
from jax import lax
from jax.experimental import pallas as pl
from jax.experimental.pallas import tpu as pltpu
from jax.experimental.pallas import tpu_sc as plsc

B = 4096
V = 100000
D = 32
N_CAT_A = 26
N_CAT_B = 13

NC = 2
NS = 16
NW = NC * NS
BPW = B // NW

VT = V // 128
VTAIL = V - VT * 128
LPC = V // 4

_mesh = plsc.VectorSubcoreMesh(core_axis_name="c", subcore_axis_name="s")
_params = pltpu.CompilerParams(
    use_tc_tiling_on_sc=True, needs_layout_passes=False)


@functools.partial(
    pl.kernel,
    out_type=(
        jax.ShapeDtypeStruct((N_CAT_A * LPC, 128), jnp.float32),
        jax.ShapeDtypeStruct((N_CAT_B * LPC, 128), jnp.float32),
    ),
    mesh=_mesh,
    scratch_types=[
        pltpu.VMEM((3, D, 512), jnp.float32),
        pltpu.VMEM((2, 128, 128), jnp.float32),
        pltpu.SemaphoreType.DMA((3,)),
        pltpu.SemaphoreType.DMA((2,)),
    ],
    compiler_params=_params,
)
def _repack(embAT, embBT, tailA, tailB, outA, outB, inb, oub, isem, wsem):
    wid = lax.axis_index("s") * NC + lax.axis_index("c")
    iota16 = lax.iota(jnp.int32, 16)

    CH = 4
    CPC = VT // CH
    VSTRAG = CPC * CH

    def transpose_lines(slot, p, n_lines):
        ssplat = jnp.full((16,), slot, jnp.int32)
        rows = [h * 16 + iota16 for h in range(2)]

        @plsc.parallel_loop(0, n_lines // 4, unroll=4)
        def per_4lines(qq):
            cbase = jnp.full((16,), 16 * qq, jnp.int32)
            for l in range(4):
                q = 4 * qq + l
                for k in range(4):
                    col = cbase + (4 * l + k)
                    for h in range(2):
                        src = plsc.load_gather(inb, [ssplat, rows[h], col])
                        oub[p, q, pl.ds(k * 32 + h * 16, 16)] = src

    def run_table(n_cols, embT, out):
        n_chunks = n_cols * CPC

        def fetch(j, slot):
            c = j // CPC
            ch = lax.rem(j, CPC)
            pltpu.async_copy(
                embT.at[c, :, pl.ds(ch * 512, 512)], inb.at[slot],
                isem.at[slot])

        def in_wait(slot):
            pltpu.make_async_copy(
                embAT.at[0, :, pl.ds(0, 512)], inb.at[slot],
                isem.at[slot]).wait()

        def wb_wait(p):
            pltpu.make_async_copy(
                outA.at[pl.ds(0, 128)], oub.at[p], wsem.at[p]).wait()

        my = (n_chunks - wid + NW - 1) // NW

        for r in range(3):
            @pl.when(r < my)
            def _(r=r):
                fetch(wid + r * NW, r)

        def body(i, carry):
            j = wid + i * NW
            slot = lax.rem(i, 3)
            p = lax.rem(i, 2)
            in_wait(slot)

            @pl.when(i >= 2)
            def _():
                wb_wait(p)

            transpose_lines(slot, p, 128)

            @pl.when(i + 3 < my)
            def _():
                fetch(j + 3 * NW, slot)

            row0 = (j // CPC) * LPC + lax.rem(j, CPC) * 128
            pltpu.async_copy(
                oub.at[p], out.at[pl.ds(row0, 128)], wsem.at[p])
            return carry

        lax.fori_loop(0, my, body, 0)

        @pl.when(my >= 1)
        def _():
            wb_wait(lax.rem(my - 1, 2))

        @pl.when(my >= 2)
        def _():
            wb_wait(lax.rem(my, 2))

    run_table(N_CAT_A, embAT, outA)
    run_table(N_CAT_B, embBT, outB)

    def straggler_tile(embT, out, c):
        pltpu.sync_copy(
            embT.at[c, :, pl.ds(VSTRAG * 128, 128)],
            inb.at[0, :, pl.ds(0, 128)])
        transpose_lines(0, 0, 32)
        pltpu.sync_copy(
            oub.at[0, pl.ds(0, 32)],
            out.at[pl.ds(c * LPC + VSTRAG * 32, 32)])

    @pl.when(wid < N_CAT_A)
    def _():
        straggler_tile(embAT, outA, wid)

    @pl.when(wid < N_CAT_B)
    def _():
        straggler_tile(embBT, outB, wid)

    def tail_tile(tail, out, c):
        nl = VTAIL // 4
        pltpu.sync_copy(tail.at[pl.ds(c * nl, nl)], oub.at[0, pl.ds(0, nl)])
        pltpu.sync_copy(
            oub.at[0, pl.ds(0, nl)],
            out.at[pl.ds(c * LPC + VT * 32, nl)])

    @pl.when(wid < N_CAT_A)
    def _():
        tail_tile(tailA, outA, wid)

    @pl.when(wid < N_CAT_B)
    def _():
        tail_tile(tailB, outB, wid)


@functools.partial(
    pl.kernel,
    out_type=(
        jax.ShapeDtypeStruct((N_CAT_A, D, B), jnp.float32),
        jax.ShapeDtypeStruct((N_CAT_A, D, B), jnp.float32),
    ),
    mesh=_mesh,
    scratch_types=[
        pltpu.VMEM((BPW,), jnp.int32),
        pltpu.VMEM((BPW,), jnp.int32),
        pltpu.VMEM((BPW, 128), jnp.float32),
        pltpu.VMEM((2, D, BPW), jnp.float32),
        pltpu.VMEM((BPW,), jnp.float32),
        pltpu.VMEM((N_CAT_B, D), jnp.float32),
        pltpu.VMEM((N_CAT_B, D), jnp.float32),
        pltpu.SemaphoreType.DMA,
        pltpu.SemaphoreType.DMA((2,)),
    ],
    compiler_params=_params,
)
def _lookup(packA, tabAT, packB, tabBT, numT, linW, linB, outAT, outBT,
            idxv, linev, gbuf, colbuf, numv, wv, bv, gsem, wsem):
    wid = lax.axis_index("s") * NC + lax.axis_index("c")
    b0 = wid * BPW
    iota16 = lax.iota(jnp.int32, 16)

    def drain_wb(p):
        pltpu.make_async_copy(
            outAT.at[0, :, pl.ds(0, BPW)], colbuf.at[p], wsem.at[p]).wait()

    def gather_phase(n_cols, pack, tabT, outT):
        def body_col(c, carry):
            p = lax.rem(c, 2)
            psplat = jnp.full((16,), p, jnp.int32)
            pltpu.sync_copy(tabT.at[c, pl.ds(b0, BPW)], idxv)
            base = c * LPC
            for bb in range(BPW // 16):
                sl = pl.ds(bb * 16, 16)
                linev[sl] = base + lax.shift_right_logical(idxv[sl], 2)
            pltpu.async_copy(pack.at[linev], gbuf, gsem)

            @pl.when(c >= 2)
            def _():
                drain_wb(p)

            pltpu.make_async_copy(
                pack.at[pl.ds(0, BPW)], gbuf, gsem).wait()

            @plsc.parallel_loop(0, BPW // 16)
            def body_bb(bb):
                qvec = (idxv[pl.ds(bb * 16, 16)] & 3) * 32
                for k in range(16):
                    i = bb * 16 + k
                    isplat = jnp.full((16,), i, jnp.int32)
                    q = qvec[k]
                    for h in range(2):
                        src = plsc.load_gather(
                            gbuf, [isplat, jnp.full((16,), q + h * 16,
                                                    jnp.int32) + iota16])
                        plsc.store_scatter(
                            colbuf,
                            [psplat, h * 16 + iota16, isplat], src)
            pltpu.async_copy(
                colbuf.at[p], outT.at[c, :, pl.ds(b0, BPW)], wsem.at[p])
            return carry

        lax.fori_loop(0, n_cols, body_col, 0)
        drain_wb(lax.rem(n_cols, 2))
        drain_wb(lax.rem(n_cols + 1, 2))

    gather_phase(N_CAT_A, packA, tabAT, outAT)
    gather_phase(N_CAT_B, packB, tabBT, outBT)

    pltpu.sync_copy(linW, wv)
    pltpu.sync_copy(linB, bv)

    def body_num(c, carry):
        p = lax.rem(c, 2)
        pltpu.sync_copy(numT.at[c, pl.ds(b0, BPW)], numv)

        @pl.when(c >= 2)
        def _():
            drain_wb(p)

        ws = [wv[c, pl.ds(0, 16)], wv[c, pl.ds(16, 16)]]
        bs = [bv[c, pl.ds(0, 16)], bv[c, pl.ds(16, 16)]]

        def body_bb(bb, inner):
            nvec = numv[pl.ds(bb * 16, 16)]
            for d in range(D):
                w_s = ws[d // 16][d % 16]
                b_s = bs[d // 16][d % 16]
                colbuf[p, d, pl.ds(bb * 16, 16)] = nvec * w_s + b_s
            return inner

        lax.fori_loop(0, BPW // 16, body_bb, 0)
        pltpu.async_copy(
            colbuf.at[p], outBT.at[N_CAT_B + c, :, pl.ds(b0, BPW)],
            wsem.at[p])
        return carry

    lax.fori_loop(0, N_CAT_B, body_num, 0)
    drain_wb(lax.rem(N_CAT_B, 2))
    drain_wb(lax.rem(N_CAT_B + 1, 2))


def kernel(tableA_cat, tableB_cat, tableB_num, embA, embB, linW, linB):
    embAT = jnp.transpose(embA, (0, 2, 1))
    embBT = jnp.transpose(embB, (0, 2, 1))
    tabAT = tableA_cat.astype(jnp.int32).T
    tabBT = tableB_cat.astype(jnp.int32).T
    numT = tableB_num.T
    tailA = embA[:, VT * 128:, :].reshape(N_CAT_A, VTAIL // 4, 128)
    tailA = tailA.reshape(N_CAT_A * (VTAIL // 4), 128)
    tailB = embB[:, VT * 128:, :].reshape(N_CAT_B, VTAIL // 4, 128)
    tailB = tailB.reshape(N_CAT_B * (VTAIL // 4), 128)
    packA, packB = _repack(embAT, embBT, tailA, tailB)
    outAT, outBT = _lookup(packA, tabAT, packB, tabBT, numT, linW, linB)
    outA = jnp.transpose(outAT, (2, 0, 1))
    outB = jnp.transpose(outBT, (2, 0, 1))
    return (outA, outB)


# speedup vs baseline: 1.1120x; 1.1120x over previous
import functools

import jax
import jax.numpy as jnp
from jax import lax
from jax.experimental import pallas as pl
from jax.experimental.pallas import tpu as pltpu
from jax.experimental.pallas import tpu_sc as plsc

B = 4096
V = 100000
D = 32
N_CAT_A = 26
N_CAT_B = 13

NC = 2
NS = 16
NW = NC * NS
BPW = B // NW

_mesh = plsc.VectorSubcoreMesh(core_axis_name="c", subcore_axis_name="s")


@functools.partial(
    pl.kernel,
    out_type=(
        jax.ShapeDtypeStruct((B, N_CAT_A, D), jnp.float32),
        jax.ShapeDtypeStruct((B, N_CAT_A, D), jnp.float32),
    ),
    mesh=_mesh,
    scratch_types=[
        pltpu.VMEM((BPW, N_CAT_A), jnp.int32),
        pltpu.VMEM((BPW, N_CAT_B), jnp.int32),
        pltpu.VMEM((BPW, N_CAT_B), jnp.float32),
        pltpu.VMEM((N_CAT_B, D), jnp.float32),
        pltpu.VMEM((N_CAT_B, D), jnp.float32),
        pltpu.VMEM((BPW,), jnp.int32),
        pltpu.VMEM((BPW, D), jnp.float32),
        pltpu.SemaphoreType.DMA,
    ],
    compiler_params=pltpu.CompilerParams(
        use_tc_tiling_on_sc=False, needs_layout_passes=False),
)
def _embed(embA, tabA, embB, tabB, num, linW, linB, outA, outB,
           idxblkA, idxblkB, numblk, wv, bv, idxv, rowv, sem):
    wid = lax.axis_index("s") * NC + lax.axis_index("c")
    b0 = wid * BPW

    pltpu.sync_copy(tabA.at[pl.ds(b0, BPW), :], idxblkA)
    pltpu.sync_copy(tabB.at[pl.ds(b0, BPW), :], idxblkB)
    pltpu.sync_copy(num.at[pl.ds(b0, BPW), :], numblk)
    pltpu.sync_copy(linW, wv)
    pltpu.sync_copy(linB, bv)

    iota16 = lax.iota(jnp.int32, 16)

    def gather_col(c, emb, idxblk, out):
        c16 = jnp.full((16,), c, dtype=jnp.int32)
        for bb in range(BPW // 16):
            i16 = bb * 16 + iota16
            idxv[pl.ds(bb * 16, 16)] = plsc.load_gather(idxblk, [i16, c16])
        pltpu.async_copy(emb.at[c].at[idxv], rowv, sem).wait()
        pltpu.sync_copy(rowv, out.at[pl.ds(b0, BPW), c])

    def body_a(c, carry):
        gather_col(c, embA, idxblkA, outA)
        return carry

    lax.fori_loop(0, N_CAT_A, body_a, 0)

    def body_b(c, carry):
        gather_col(c, embB, idxblkB, outB)
        return carry

    lax.fori_loop(0, N_CAT_B, body_b, 0)

    def body_num(c, carry):
        c16 = jnp.full((16,), c, dtype=jnp.int32)
        w0 = wv[c, pl.ds(0, 16)]
        w1 = wv[c, pl.ds(16, 16)]
        v0 = bv[c, pl.ds(0, 16)]
        v1 = bv[c, pl.ds(16, 16)]

        def body_row(bb, inner):
            nums = plsc.load_gather(numblk, [bb * 16 + iota16, c16])
            for k in range(16):
                s = nums[k]
                rowv[bb * 16 + k, pl.ds(0, 16)] = s * w0 + v0
                rowv[bb * 16 + k, pl.ds(16, 16)] = s * w1 + v1
            return inner

        lax.fori_loop(0, BPW // 16, body_row, 0)
        pltpu.sync_copy(rowv, outB.at[pl.ds(b0, BPW), N_CAT_B + c])
        return carry

    lax.fori_loop(0, N_CAT_B, body_num, 0)


def kernel(tableA_cat, tableB_cat, tableB_num, embA, embB, linW, linB):
    outA, outB = _embed(embA, tableA_cat.astype(jnp.int32), embB,
                        tableB_cat.astype(jnp.int32), tableB_num, linW, linB)
    return (outA, outB)
